# baseline (speedup 1.0000x reference)
import functools

import jax
import jax.numpy as jnp
from jax import lax
from jax.experimental import pallas as pl
from jax.experimental.pallas import tpu as pltpu
from jax.experimental.pallas import tpu_sc as plsc

D_VOCAB = 1000000
D_EMB = 32
NW = 32


def _mesh():
    return plsc.VectorSubcoreMesh(core_axis_name="c", subcore_axis_name="s")


def _wid():
    return lax.axis_index("s") * 2 + lax.axis_index("c")


WRM_ROWS = (D_VOCAB // 128 + 1) * 32


def _reformat_table(weight_t, weight_tail):
    full_units = D_VOCAB // 128
    units = full_units + 1
    per_w = (units + NW - 1) // NW

    @functools.partial(
        pl.kernel,
        mesh=_mesh(),
        out_type=jax.ShapeDtypeStruct((WRM_ROWS, 128), jnp.float32),
        compiler_params=pltpu.CompilerParams(
            use_tc_tiling_on_sc=True, needs_layout_passes=False),
        scratch_types=[
            pltpu.VMEM((D_EMB, 128), jnp.float32),
            pltpu.VMEM((32, 128), jnp.float32),
        ],
    )
    def ka(wt_hbm, wtail_hbm, wrm_hbm, src_v, stage_v):
        w = _wid()
        iota = lax.iota(jnp.int32, 16)

        def transpose_and_store(u):
            for v in range(128):
                col = jnp.full((16,), v, jnp.int32)
                for k in range(2):
                    vec = plsc.load_gather(src_v, [iota + 16 * k, col])
                    p0 = v * 32 + 16 * k
                    stage_v[p0 // 128, pl.ds(p0 % 128, 16)] = vec
            pltpu.sync_copy(stage_v, wrm_hbm.at[pl.ds(u * 32, 32)])

        def body(i, carry):
            u = i * NW + w

            @pl.when(u < full_units)
            def _():
                pltpu.sync_copy(wt_hbm.at[:, pl.ds(u * 128, 128)], src_v)
                transpose_and_store(u)

            @pl.when(u == full_units)
            def _():
                pltpu.sync_copy(wtail_hbm, src_v)
                transpose_and_store(u)

            return carry

        lax.fori_loop(0, per_w, body, 0)

    return ka(weight_t, weight_tail)


def _gather_native(idx4, w_lin):
    n_sb, n_bt = idx4.shape[0], idx4.shape[1]
    per_w = (n_sb * n_bt) // NW

    @functools.partial(
        pl.kernel,
        mesh=_mesh(),
        out_type=jax.ShapeDtypeStruct((200, 4, 32, 8, 128), jnp.float32),
        compiler_params=pltpu.CompilerParams(
            use_tc_tiling_on_sc=False, needs_layout_passes=False),
        scratch_types=[
            pltpu.VMEM((8, 128), jnp.int32),
            pltpu.VMEM((1024, D_EMB), jnp.float32),
            pltpu.VMEM((8, 4, 8, 128), jnp.float32),
            pltpu.SemaphoreType.DMA,
            pltpu.SemaphoreType.DMA,
        ],
    )
    def kb(idx_hbm, wrm_hbm, out_hbm, idx_v, rows_v, stage_v, sem_g, sem_s):
        w = _wid()
        iota = lax.iota(jnp.int32, 16)

        def body(i, carry):
            unit = w * per_w + i
            sb = unit // n_bt
            bt = unit % n_bt

            pltpu.sync_copy(idx_hbm.at[sb, bt], idx_v)
            for s8 in range(8):
                pltpu.async_copy(
                    wrm_hbm.at[idx_v.at[s8]],
                    rows_v.at[pl.ds(s8 * 128, 128)],
                    sem_g,
                )
            for s8 in range(8):
                pltpu.make_async_copy(
                    wrm_hbm.at[idx_v.at[s8]],
                    rows_v.at[pl.ds(s8 * 128, 128)],
                    sem_g,
                ).wait()

            for s8 in range(8):
                for E in range(4):
                    def tbody(e8, c):
                        col = jnp.full((16,), E * 8 + e8, jnp.int32)
                        for j in range(8):
                            row = iota + (s8 * 128 + j * 16)
                            vec = plsc.load_gather(rows_v, [row, col])
                            stage_v[s8, E, e8, pl.ds(j * 16, 16)] = vec
                        return c

                    lax.fori_loop(0, 8, tbody, 0)

            for s8 in range(8):
                for E in range(4):
                    pltpu.async_copy(
                        stage_v.at[s8, E],
                        out_hbm.at[sb * 8 + s8, E, bt],
                        sem_s,
                    )
            for s8 in range(8):
                for E in range(4):
                    pltpu.make_async_copy(
                        stage_v.at[s8, E],
                        out_hbm.at[sb * 8 + s8, E, bt],
                        sem_s,
                    ).wait()
            return carry

        lax.fori_loop(0, per_w, body, 0)

    return kb(idx4, w_lin)


def kernel(input, weight):
    idx4 = (input.astype(jnp.int32).T
            .reshape(25, 8, 32, 128).transpose(0, 2, 1, 3))
    wtail = jnp.pad(weight[D_VOCAB - 64:, :].T, ((0, 0), (0, 64)))
    w_rm = _reformat_table(weight.T, wtail)
    w_lin = w_rm.reshape(WRM_ROWS * 4, D_EMB)
    out5 = _gather_native(idx4, w_lin)
    return out5.transpose(2, 4, 0, 1, 3).reshape(4096, 200, 32)


# speedup vs baseline: 1.2465x; 1.2465x over previous
import functools

import jax
import jax.numpy as jnp
from jax import lax
from jax.experimental import pallas as pl
from jax.experimental.pallas import tpu as pltpu
from jax.experimental.pallas import tpu_sc as plsc

D_VOCAB = 1000000
D_EMB = 32
NW = 32
WRM_ROWS = (D_VOCAB // 128 + 1) * 32


def _mesh():
    return plsc.VectorSubcoreMesh(core_axis_name="c", subcore_axis_name="s")


def _wid():
    return lax.axis_index("s") * 2 + lax.axis_index("c")


def _reformat_table(weight_t, weight_tail):
    full_units = D_VOCAB // 128
    units = full_units + 1
    per_w = 246

    @functools.partial(
        pl.kernel,
        mesh=_mesh(),
        out_type=jax.ShapeDtypeStruct((WRM_ROWS, 128), jnp.float32),
        compiler_params=pltpu.CompilerParams(
            use_tc_tiling_on_sc=True, needs_layout_passes=False),
        scratch_types=[
            pltpu.VMEM((D_EMB, 128), jnp.float32),
            pltpu.VMEM((D_EMB, 128), jnp.float32),
            pltpu.VMEM((32, 128), jnp.float32),
            pltpu.VMEM((32, 128), jnp.float32),
            pltpu.SemaphoreType.DMA,
            pltpu.SemaphoreType.DMA,
            pltpu.SemaphoreType.DMA,
            pltpu.SemaphoreType.DMA,
        ],
    )
    def ka(wt_hbm, wtail_hbm, wrm_hbm, src0, src1, st0, st1,
           sa0, sa1, sw0, sw1):
        srcs, stages = (src0, src1), (st0, st1)
        semas, semws = (sa0, sa1), (sw0, sw1)
        w = _wid()
        iota = lax.iota(jnp.int32, 16)

        def u_of(i):
            return i * NW + w

        def fire_load(i, p):
            u = u_of(i)

            @pl.when(u < full_units)
            def _():
                pltpu.async_copy(
                    wt_hbm.at[:, pl.ds(u * 128, 128)], srcs[p], semas[p])

            @pl.when(u == full_units)
            def _():
                pltpu.async_copy(wtail_hbm, srcs[p], semas[p])

        def wait_load(i, p):
            @pl.when(u_of(i) < units)
            def _():
                pltpu.make_async_copy(wtail_hbm, srcs[p], semas[p]).wait()

        def transpose(p):
            def tb(t, c):
                for u4 in range(4):
                    col = jnp.full((16,), t * 4 + u4, jnp.int32)
                    for k in range(2):
                        vec = plsc.load_gather(srcs[p], [iota + 16 * k, col])
                        stages[p][t, pl.ds(u4 * 32 + 16 * k, 16)] = vec
                return c

            lax.fori_loop(0, 32, tb, 0)

        def fire_write(i, p):
            u = u_of(i)

            @pl.when(u < units)
            def _():
                pltpu.async_copy(
                    stages[p], wrm_hbm.at[pl.ds(u * 32, 32)], semws[p])

        def wait_write(i, p):
            @pl.when((i >= 0) & (u_of(i) < units))
            def _():
                pltpu.make_async_copy(
                    stages[p], wrm_hbm.at[pl.ds(0, 32)], semws[p]).wait()

        def step(i, p):
            fire_load(i + 1, p ^ 1)
            wait_load(i, p)
            wait_write(i - 2, p)
            transpose(p)
            fire_write(i, p)

        fire_load(0, 0)

        def body(r, carry):
            step(2 * r, 0)
            step(2 * r + 1, 1)
            return carry

        lax.fori_loop(0, per_w // 2, body, 0)
        wait_write(per_w - 2, 0)
        wait_write(per_w - 1, 1)

    return ka(weight_t, weight_tail)


def _gather_native(idx4, w_lin):
    per_w = 50

    @functools.partial(
        pl.kernel,
        mesh=_mesh(),
        out_type=jax.ShapeDtypeStruct((200, 4, 32, 8, 128), jnp.float32),
        compiler_params=pltpu.CompilerParams(
            use_tc_tiling_on_sc=False, needs_layout_passes=False),
        scratch_types=[
            pltpu.VMEM((4, 128), jnp.int32),
            pltpu.VMEM((4, 128), jnp.int32),
            pltpu.VMEM((512, D_EMB), jnp.float32),
            pltpu.VMEM((512, D_EMB), jnp.float32),
            pltpu.VMEM((4, 4, 8, 128), jnp.float32),
            pltpu.VMEM((4, 4, 8, 128), jnp.float32),
            pltpu.SemaphoreType.DMA,
            pltpu.SemaphoreType.DMA,
            pltpu.SemaphoreType.DMA,
            pltpu.SemaphoreType.DMA,
            pltpu.SemaphoreType.DMA,
            pltpu.SemaphoreType.DMA,
        ],
    )
    def kb(idx_hbm, wrm_hbm, out_hbm, idx0, idx1, rows0, rows1, st0, st1,
           si0, si1, sg0, sg1, ss0, ss1):
        idxs, rowss, stages = (idx0, idx1), (rows0, rows1), (st0, st1)
        semis, semgs, semss = (si0, si1), (sg0, sg1), (ss0, ss1)
        w = _wid()
        iota = lax.iota(jnp.int32, 16)

        def coords(i):
            unit = w * per_w + i
            sb = unit // 64
            rem = unit % 64
            return sb, rem // 2, rem % 2

        def fire_idx(i, p):
            sb, bt, h = coords(i)
            pltpu.async_copy(
                idx_hbm.at[sb, bt, pl.ds(h * 4, 4)], idxs[p], semis[p])

        def wait_idx(p):
            pltpu.make_async_copy(
                idx_hbm.at[0, 0, pl.ds(0, 4)], idxs[p], semis[p]).wait()

        def fire_gathers(p):
            for s8 in range(4):
                pltpu.async_copy(
                    wrm_hbm.at[idxs[p].at[s8]],
                    rowss[p].at[pl.ds(s8 * 128, 128)],
                    semgs[p])

        def wait_gathers(p):
            for s8 in range(4):
                pltpu.make_async_copy(
                    wrm_hbm.at[idxs[p].at[s8]],
                    rowss[p].at[pl.ds(s8 * 128, 128)],
                    semgs[p]).wait()

        def transpose(p):
            for s8 in range(4):
                for E in range(4):
                    def tbody(e8, c):
                        col = jnp.full((16,), E * 8 + e8, jnp.int32)
                        for j in range(8):
                            row = iota + (s8 * 128 + j * 16)
                            vec = plsc.load_gather(rowss[p], [row, col])
                            stages[p][s8, E, e8, pl.ds(j * 16, 16)] = vec
                        return c

                    lax.fori_loop(0, 8, tbody, 0)

        def fire_writes(i, p):
            sb, bt, h = coords(i)
            for s8 in range(4):
                for E in range(4):
                    pltpu.async_copy(
                        stages[p].at[s8, E],
                        out_hbm.at[sb * 8 + h * 4 + s8, E, bt],
                        semss[p])

        def wait_writes(p):
            for s8 in range(4):
                for E in range(4):
                    pltpu.make_async_copy(
                        stages[p].at[s8, E], out_hbm.at[0, 0, 0],
                        semss[p]).wait()

        def step(i, p):
            wait_gathers(p)

            @pl.when(i + 2 < per_w)
            def _():
                fire_idx(i + 2, p)

            @pl.when(i + 1 < per_w)
            def _():
                wait_idx(p ^ 1)
                fire_gathers(p ^ 1)

            @pl.when(i >= 2)
            def _():
                wait_writes(p)

            transpose(p)
            fire_writes(i, p)

        fire_idx(0, 0)
        fire_idx(1, 1)
        wait_idx(0)
        fire_gathers(0)

        def body(r, carry):
            step(2 * r, 0)
            step(2 * r + 1, 1)
            return carry

        lax.fori_loop(0, per_w // 2, body, 0)
        wait_writes(0)
        wait_writes(1)

    return kb(idx4, w_lin)


def kernel(input, weight):
    idx4 = (input.astype(jnp.int32).T
            .reshape(25, 8, 32, 128).transpose(0, 2, 1, 3))
    wtail = jnp.pad(weight[D_VOCAB - 64:, :].T, ((0, 0), (0, 64)))
    w_rm = _reformat_table(weight.T, wtail)
    w_lin = w_rm.reshape(WRM_ROWS * 4, D_EMB)
    out5 = _gather_native(idx4, w_lin)
    return out5.transpose(2, 4, 0, 1, 3).reshape(4096, 200, 32)


# speedup vs baseline: 1.5299x; 1.2274x over previous
import functools

import jax
import jax.numpy as jnp
from jax import lax
from jax.experimental import pallas as pl
from jax.experimental.pallas import tpu as pltpu
from jax.experimental.pallas import tpu_sc as plsc

D_VOCAB = 1000000
D_EMB = 32
NW = 32
WRM_ROWS = (D_VOCAB // 128 + 1) * 32


def _mesh():
    return plsc.VectorSubcoreMesh(core_axis_name="c", subcore_axis_name="s")


def _wid():
    return lax.axis_index("s") * 2 + lax.axis_index("c")


def _reformat_table(weight_t, weight_tail):
    full_units = D_VOCAB // 128
    units = full_units + 1
    per_w = 246

    @functools.partial(
        pl.kernel,
        mesh=_mesh(),
        out_type=jax.ShapeDtypeStruct((WRM_ROWS, 128), jnp.float32),
        compiler_params=pltpu.CompilerParams(
            use_tc_tiling_on_sc=True, needs_layout_passes=False),
        scratch_types=[
            pltpu.VMEM((D_EMB, 128), jnp.float32),
            pltpu.VMEM((D_EMB, 128), jnp.float32),
            pltpu.VMEM((32, 128), jnp.float32),
            pltpu.VMEM((32, 128), jnp.float32),
            pltpu.SemaphoreType.DMA,
            pltpu.SemaphoreType.DMA,
            pltpu.SemaphoreType.DMA,
            pltpu.SemaphoreType.DMA,
        ],
    )
    def ka(wt_hbm, wtail_hbm, wrm_hbm, src0, src1, st0, st1,
           sa0, sa1, sw0, sw1):
        srcs, stages = (src0, src1), (st0, st1)
        semas, semws = (sa0, sa1), (sw0, sw1)
        w = _wid()
        iota = lax.iota(jnp.int32, 16)

        def u_of(i):
            return i * NW + w

        def fire_load(i, p):
            u = u_of(i)

            @pl.when(u < full_units)
            def _():
                pltpu.async_copy(
                    wt_hbm.at[:, pl.ds(u * 128, 128)], srcs[p], semas[p])

            @pl.when(u == full_units)
            def _():
                pltpu.async_copy(wtail_hbm, srcs[p], semas[p])

        def wait_load(i, p):
            @pl.when(u_of(i) < units)
            def _():
                pltpu.make_async_copy(wtail_hbm, srcs[p], semas[p]).wait()

        row_base = lax.shift_right_logical(iota, 2)
        col_pat = (iota & 3) * 32

        def transpose(p):
            def tb(eb, c):
                for de in range(4):
                    e = eb * 4 + de
                    col_vec = col_pat + e
                    for j in range(8):
                        vec = srcs[p][e, pl.ds(j * 16, 16)]
                        plsc.store_scatter(
                            stages[p], [row_base + j * 4, col_vec], vec)
                return c

            lax.fori_loop(0, 8, tb, 0)

        def fire_write(i, p):
            u = u_of(i)

            @pl.when(u < units)
            def _():
                pltpu.async_copy(
                    stages[p], wrm_hbm.at[pl.ds(u * 32, 32)], semws[p])

        def wait_write(i, p):
            @pl.when((i >= 0) & (u_of(i) < units))
            def _():
                pltpu.make_async_copy(
                    stages[p], wrm_hbm.at[pl.ds(0, 32)], semws[p]).wait()

        def step(i, p):
            fire_load(i + 1, p ^ 1)
            wait_load(i, p)
            wait_write(i - 2, p)
            transpose(p)
            fire_write(i, p)

        fire_load(0, 0)

        def body(r, carry):
            step(2 * r, 0)
            step(2 * r + 1, 1)
            return carry

        lax.fori_loop(0, per_w // 2, body, 0)
        wait_write(per_w - 2, 0)
        wait_write(per_w - 1, 1)

    return ka(weight_t, weight_tail)


def _gather_native(idx4, w_lin):
    per_w = 50

    @functools.partial(
        pl.kernel,
        mesh=_mesh(),
        out_type=jax.ShapeDtypeStruct((200, 4, 32, 8, 128), jnp.float32),
        compiler_params=pltpu.CompilerParams(
            use_tc_tiling_on_sc=False, needs_layout_passes=False),
        scratch_types=[
            pltpu.VMEM((4, 128), jnp.int32),
            pltpu.VMEM((4, 128), jnp.int32),
            pltpu.VMEM((512, D_EMB), jnp.float32),
            pltpu.VMEM((512, D_EMB), jnp.float32),
            pltpu.VMEM((4, 4, 8, 128), jnp.float32),
            pltpu.VMEM((4, 4, 8, 128), jnp.float32),
            pltpu.SemaphoreType.DMA,
            pltpu.SemaphoreType.DMA,
            pltpu.SemaphoreType.DMA,
            pltpu.SemaphoreType.DMA,
            pltpu.SemaphoreType.DMA,
            pltpu.SemaphoreType.DMA,
        ],
    )
    def kb(idx_hbm, wrm_hbm, out_hbm, idx0, idx1, rows0, rows1, st0, st1,
           si0, si1, sg0, sg1, ss0, ss1):
        idxs, rowss, stages = (idx0, idx1), (rows0, rows1), (st0, st1)
        semis, semgs, semss = (si0, si1), (sg0, sg1), (ss0, ss1)
        w = _wid()
        iota = lax.iota(jnp.int32, 16)

        def coords(i):
            unit = w * per_w + i
            sb = unit // 64
            rem = unit % 64
            return sb, rem // 2, rem % 2

        def fire_idx(i, p):
            sb, bt, h = coords(i)
            pltpu.async_copy(
                idx_hbm.at[sb, bt, pl.ds(h * 4, 4)], idxs[p], semis[p])

        def wait_idx(p):
            pltpu.make_async_copy(
                idx_hbm.at[0, 0, pl.ds(0, 4)], idxs[p], semis[p]).wait()

        def fire_gathers(p):
            for s8 in range(4):
                pltpu.async_copy(
                    wrm_hbm.at[idxs[p].at[s8]],
                    rowss[p].at[pl.ds(s8 * 128, 128)],
                    semgs[p])

        def wait_gathers(p):
            for s8 in range(4):
                pltpu.make_async_copy(
                    wrm_hbm.at[idxs[p].at[s8]],
                    rowss[p].at[pl.ds(s8 * 128, 128)],
                    semgs[p]).wait()

        e_hi = [lax.shift_right_logical(iota + 16 * k, 3) for k in range(2)]
        e_lo = [(iota + 16 * k) & 7 for k in range(2)]

        def transpose(p):
            for s8 in range(4):
                s8_vec = jnp.full((16,), s8, jnp.int32)

                def tbody(rb, c):
                    for dr in range(16):
                        b = rb * 16 + dr
                        b_vec = jnp.full((16,), b, jnp.int32)
                        r = s8 * 128 + b
                        for k in range(2):
                            vec = rowss[p][r, pl.ds(16 * k, 16)]
                            plsc.store_scatter(
                                stages[p], [s8_vec, e_hi[k], e_lo[k], b_vec],
                                vec)
                    return c

                lax.fori_loop(0, 8, tbody, 0)

        def fire_writes(i, p):
            sb, bt, h = coords(i)
            for s8 in range(4):
                for E in range(4):
                    pltpu.async_copy(
                        stages[p].at[s8, E],
                        out_hbm.at[sb * 8 + h * 4 + s8, E, bt],
                        semss[p])

        def wait_writes(p):
            for s8 in range(4):
                for E in range(4):
                    pltpu.make_async_copy(
                        stages[p].at[s8, E], out_hbm.at[0, 0, 0],
                        semss[p]).wait()

        def step(i, p):
            wait_gathers(p)

            @pl.when(i + 2 < per_w)
            def _():
                fire_idx(i + 2, p)

            @pl.when(i + 1 < per_w)
            def _():
                wait_idx(p ^ 1)
                fire_gathers(p ^ 1)

            @pl.when(i >= 2)
            def _():
                wait_writes(p)

            transpose(p)
            fire_writes(i, p)

        fire_idx(0, 0)
        fire_idx(1, 1)
        wait_idx(0)
        fire_gathers(0)

        def body(r, carry):
            step(2 * r, 0)
            step(2 * r + 1, 1)
            return carry

        lax.fori_loop(0, per_w // 2, body, 0)
        wait_writes(0)
        wait_writes(1)

    return kb(idx4, w_lin)


def kernel(input, weight):
    idx4 = (input.astype(jnp.int32).T
            .reshape(25, 8, 32, 128).transpose(0, 2, 1, 3))
    wtail = jnp.pad(weight[D_VOCAB - 64:, :].T, ((0, 0), (0, 64)))
    w_rm = _reformat_table(weight.T, wtail)
    w_lin = w_rm.reshape(WRM_ROWS * 4, D_EMB)
    out5 = _gather_native(idx4, w_lin)
    return out5.transpose(2, 4, 0, 1, 3).reshape(4096, 200, 32)


# speedup vs baseline: 1.5414x; 1.0075x over previous
import functools

import jax
import jax.numpy as jnp
from jax import lax
from jax.experimental import pallas as pl
from jax.experimental.pallas import tpu as pltpu
from jax.experimental.pallas import tpu_sc as plsc

D_VOCAB = 1000000
D_EMB = 32
NW = 32
WRM_ROWS = (D_VOCAB // 128 + 1) * 32


def _mesh():
    return plsc.VectorSubcoreMesh(core_axis_name="c", subcore_axis_name="s")


def _wid():
    return lax.axis_index("s") * 2 + lax.axis_index("c")


def _reformat_table(weight_t, weight_tail):
    full_units = D_VOCAB // 128
    units = full_units + 1
    per_w = 246

    @functools.partial(
        pl.kernel,
        mesh=_mesh(),
        out_type=jax.ShapeDtypeStruct((WRM_ROWS * 128,), jnp.float32),
        compiler_params=pltpu.CompilerParams(
            use_tc_tiling_on_sc=True, needs_layout_passes=False),
        scratch_types=[
            pltpu.VMEM((D_EMB, 128), jnp.float32),
            pltpu.VMEM((D_EMB, 128), jnp.float32),
            pltpu.VMEM((4096,), jnp.float32),
            pltpu.VMEM((4096,), jnp.float32),
            pltpu.SemaphoreType.DMA,
            pltpu.SemaphoreType.DMA,
            pltpu.SemaphoreType.DMA,
            pltpu.SemaphoreType.DMA,
        ],
    )
    def ka(wt_hbm, wtail_hbm, wrm_hbm, src0, src1, st0, st1,
           sa0, sa1, sw0, sw1):
        srcs, stages = (src0, src1), (st0, st1)
        semas, semws = (sa0, sa1), (sw0, sw1)
        w = _wid()
        iota = lax.iota(jnp.int32, 16)

        def u_of(i):
            return i * NW + w

        def fire_load(i, p):
            u = u_of(i)

            @pl.when(u < full_units)
            def _():
                pltpu.async_copy(
                    wt_hbm.at[:, pl.ds(u * 128, 128)], srcs[p], semas[p])

            @pl.when(u == full_units)
            def _():
                pltpu.async_copy(wtail_hbm, srcs[p], semas[p])

        def wait_load(i, p):
            @pl.when(u_of(i) < units)
            def _():
                pltpu.make_async_copy(wtail_hbm, srcs[p], semas[p]).wait()

        jconst = [iota * 32 + j * 512 for j in range(8)]

        def transpose(p):
            def tb(eb, c):
                for de in range(4):
                    e = eb * 4 + de
                    e_vec = jnp.full((16,), e, jnp.int32)
                    for j in range(8):
                        vec = srcs[p][e, pl.ds(j * 16, 16)]
                        plsc.store_scatter(stages[p], [jconst[j] + e_vec], vec)
                return c

            lax.fori_loop(0, 8, tb, 0)

        def fire_write(i, p):
            u = u_of(i)

            @pl.when(u < units)
            def _():
                pltpu.async_copy(
                    stages[p], wrm_hbm.at[pl.ds(u * 4096, 4096)], semws[p])

        def wait_write(i, p):
            @pl.when((i >= 0) & (u_of(i) < units))
            def _():
                pltpu.make_async_copy(
                    stages[p], wrm_hbm.at[pl.ds(0, 4096)], semws[p]).wait()

        def step(i, p):
            fire_load(i + 1, p ^ 1)
            wait_load(i, p)
            wait_write(i - 2, p)
            transpose(p)
            fire_write(i, p)

        fire_load(0, 0)

        def body(r, carry):
            step(2 * r, 0)
            step(2 * r + 1, 1)
            return carry

        lax.fori_loop(0, per_w // 2, body, 0)
        wait_write(per_w - 2, 0)
        wait_write(per_w - 1, 1)

    return ka(weight_t, weight_tail)


def _gather_native(idx4, w_lin):
    per_w = 50

    @functools.partial(
        pl.kernel,
        mesh=_mesh(),
        out_type=jax.ShapeDtypeStruct((200, 4, 32, 1024), jnp.float32),
        compiler_params=pltpu.CompilerParams(
            use_tc_tiling_on_sc=False, needs_layout_passes=False),
        scratch_types=[
            pltpu.VMEM((4, 128), jnp.int32),
            pltpu.VMEM((4, 128), jnp.int32),
            pltpu.VMEM((512, D_EMB), jnp.float32),
            pltpu.VMEM((512, D_EMB), jnp.float32),
            pltpu.VMEM((16384,), jnp.float32),
            pltpu.VMEM((16384,), jnp.float32),
            pltpu.SemaphoreType.DMA,
            pltpu.SemaphoreType.DMA,
            pltpu.SemaphoreType.DMA,
            pltpu.SemaphoreType.DMA,
            pltpu.SemaphoreType.DMA,
            pltpu.SemaphoreType.DMA,
        ],
    )
    def kb(idx_hbm, wrm_hbm, out_hbm, idx0, idx1, rows0, rows1, st0, st1,
           si0, si1, sg0, sg1, ss0, ss1):
        idxs, rowss, stages = (idx0, idx1), (rows0, rows1), (st0, st1)
        semis, semgs, semss = (si0, si1), (sg0, sg1), (ss0, ss1)
        w = _wid()
        iota = lax.iota(jnp.int32, 16)

        def coords(i):
            unit = w * per_w + i
            sb = unit // 64
            rem = unit % 64
            return sb, rem // 2, rem % 2

        def fire_idx(i, p):
            sb, bt, h = coords(i)
            pltpu.async_copy(
                idx_hbm.at[sb, bt, pl.ds(h * 4, 4)], idxs[p], semis[p])

        def wait_idx(p):
            pltpu.make_async_copy(
                idx_hbm.at[0, 0, pl.ds(0, 4)], idxs[p], semis[p]).wait()

        def fire_gathers(p):
            for s8 in range(4):
                pltpu.async_copy(
                    wrm_hbm.at[idxs[p].at[s8]],
                    rowss[p].at[pl.ds(s8 * 128, 128)],
                    semgs[p])

        def wait_gathers(p):
            for s8 in range(4):
                pltpu.make_async_copy(
                    wrm_hbm.at[idxs[p].at[s8]],
                    rowss[p].at[pl.ds(s8 * 128, 128)],
                    semgs[p]).wait()

        skc = [[(iota + 16 * k) * 128 + s8 * 4096 for k in range(2)]
               for s8 in range(4)]

        def transpose(p):
            for s8 in range(4):
                def tbody(rb, c):
                    for dr in range(16):
                        b = rb * 16 + dr
                        b_vec = jnp.full((16,), b, jnp.int32)
                        r = s8 * 128 + b
                        for k in range(2):
                            vec = rowss[p][r, pl.ds(16 * k, 16)]
                            plsc.store_scatter(
                                stages[p], [skc[s8][k] + b_vec], vec)
                    return c

                lax.fori_loop(0, 8, tbody, 0)

        def fire_writes(i, p):
            sb, bt, h = coords(i)
            for s8 in range(4):
                for E in range(4):
                    pltpu.async_copy(
                        stages[p].at[pl.ds((s8 * 4 + E) * 1024, 1024)],
                        out_hbm.at[sb * 8 + h * 4 + s8, E, bt],
                        semss[p])

        def wait_writes(p):
            for s8 in range(4):
                for E in range(4):
                    pltpu.make_async_copy(
                        stages[p].at[pl.ds((s8 * 4 + E) * 1024, 1024)],
                        out_hbm.at[0, 0, 0], semss[p]).wait()

        def step(i, p):
            wait_gathers(p)

            @pl.when(i + 2 < per_w)
            def _():
                fire_idx(i + 2, p)

            @pl.when(i + 1 < per_w)
            def _():
                wait_idx(p ^ 1)
                fire_gathers(p ^ 1)

            @pl.when(i >= 2)
            def _():
                wait_writes(p)

            transpose(p)
            fire_writes(i, p)

        fire_idx(0, 0)
        fire_idx(1, 1)
        wait_idx(0)
        fire_gathers(0)

        def body(r, carry):
            step(2 * r, 0)
            step(2 * r + 1, 1)
            return carry

        lax.fori_loop(0, per_w // 2, body, 0)
        wait_writes(0)
        wait_writes(1)

    return kb(idx4, w_lin)


def kernel(input, weight):
    idx4 = (input.astype(jnp.int32).T
            .reshape(25, 8, 32, 128).transpose(0, 2, 1, 3))
    wtail = jnp.pad(weight[D_VOCAB - 64:, :].T, ((0, 0), (0, 64)))
    w_rm = _reformat_table(weight.T, wtail)
    w_lin = w_rm.reshape(WRM_ROWS * 4, D_EMB)
    out4 = _gather_native(idx4, w_lin)
    out5 = out4.reshape(200, 4, 32, 8, 128)
    return out5.transpose(2, 4, 0, 1, 3).reshape(4096, 200, 32)


# speedup vs baseline: 1.6363x; 1.0616x over previous
import functools

import jax
import jax.numpy as jnp
from jax import lax
from jax.experimental import pallas as pl
from jax.experimental.pallas import tpu as pltpu
from jax.experimental.pallas import tpu_sc as plsc

D_VOCAB = 1000000
D_EMB = 32
NW = 32
WRM_ROWS = (D_VOCAB // 128 + 1) * 32


def _mesh():
    return plsc.VectorSubcoreMesh(core_axis_name="c", subcore_axis_name="s")


def _wid():
    return lax.axis_index("s") * 2 + lax.axis_index("c")


def _reformat_table(weight_t, weight_tail):
    full_units = D_VOCAB // 128
    units = full_units + 1
    per_w = 246

    @functools.partial(
        pl.kernel,
        mesh=_mesh(),
        out_type=jax.ShapeDtypeStruct((WRM_ROWS * 128,), jnp.float32),
        compiler_params=pltpu.CompilerParams(
            use_tc_tiling_on_sc=True, needs_layout_passes=False),
        scratch_types=[
            pltpu.VMEM((D_EMB, 128), jnp.float32),
            pltpu.VMEM((D_EMB, 128), jnp.float32),
            pltpu.VMEM((4096,), jnp.float32),
            pltpu.VMEM((4096,), jnp.float32),
            pltpu.SemaphoreType.DMA,
            pltpu.SemaphoreType.DMA,
            pltpu.SemaphoreType.DMA,
            pltpu.SemaphoreType.DMA,
        ],
    )
    def ka(wt_hbm, wtail_hbm, wrm_hbm, src0, src1, st0, st1,
           sa0, sa1, sw0, sw1):
        srcs, stages = (src0, src1), (st0, st1)
        semas, semws = (sa0, sa1), (sw0, sw1)
        w = _wid()
        iota = lax.iota(jnp.int32, 16)

        def u_of(i):
            return i * NW + w

        def fire_load(i, p):
            u = u_of(i)

            @pl.when(u < full_units)
            def _():
                pltpu.async_copy(
                    wt_hbm.at[:, pl.ds(u * 128, 128)], srcs[p], semas[p])

            @pl.when(u == full_units)
            def _():
                pltpu.async_copy(wtail_hbm, srcs[p], semas[p])

        def wait_load(i, p):
            @pl.when(u_of(i) < units)
            def _():
                pltpu.make_async_copy(wtail_hbm, srcs[p], semas[p]).wait()

        jconst = [iota * 32 + j * 512 for j in range(8)]

        def transpose(p):
            def tb(eb, c):
                for de2 in range(2):
                    pairs = []
                    for de in range(2):
                        e = eb * 4 + de2 * 2 + de
                        e_vec = jnp.full((16,), e, jnp.int32)
                        for j in range(8):
                            vec = srcs[p][e, pl.ds(j * 16, 16)]
                            pairs.append((jconst[j] + e_vec, vec))
                    for idxv, vec in pairs:
                        plsc.store_scatter(stages[p], [idxv], vec)
                return c

            lax.fori_loop(0, 8, tb, 0)

        def fire_write(i, p):
            u = u_of(i)

            @pl.when(u < units)
            def _():
                pltpu.async_copy(
                    stages[p], wrm_hbm.at[pl.ds(u * 4096, 4096)], semws[p])

        def wait_write(i, p):
            @pl.when((i >= 0) & (u_of(i) < units))
            def _():
                pltpu.make_async_copy(
                    stages[p], wrm_hbm.at[pl.ds(0, 4096)], semws[p]).wait()

        def step(i, p):
            fire_load(i + 1, p ^ 1)
            wait_load(i, p)
            wait_write(i - 2, p)
            transpose(p)
            fire_write(i, p)

        fire_load(0, 0)

        def body(r, carry):
            step(2 * r, 0)
            step(2 * r + 1, 1)
            return carry

        lax.fori_loop(0, per_w // 2, body, 0)
        wait_write(per_w - 2, 0)
        wait_write(per_w - 1, 1)

    return ka(weight_t, weight_tail)


def _gather_native(idx4, w_lin):
    per_w = 50

    @functools.partial(
        pl.kernel,
        mesh=_mesh(),
        out_type=jax.ShapeDtypeStruct((200, 4, 32, 1024), jnp.float32),
        compiler_params=pltpu.CompilerParams(
            use_tc_tiling_on_sc=False, needs_layout_passes=False),
        scratch_types=[
            pltpu.VMEM((4, 128), jnp.int32),
            pltpu.VMEM((4, 128), jnp.int32),
            pltpu.VMEM((512, D_EMB), jnp.float32),
            pltpu.VMEM((512, D_EMB), jnp.float32),
            pltpu.VMEM((16384,), jnp.float32),
            pltpu.VMEM((16384,), jnp.float32),
            pltpu.SemaphoreType.DMA,
            pltpu.SemaphoreType.DMA,
            pltpu.SemaphoreType.DMA,
            pltpu.SemaphoreType.DMA,
            pltpu.SemaphoreType.DMA,
            pltpu.SemaphoreType.DMA,
        ],
    )
    def kb(idx_hbm, wrm_hbm, out_hbm, idx0, idx1, rows0, rows1, st0, st1,
           si0, si1, sg0, sg1, ss0, ss1):
        idxs, rowss, stages = (idx0, idx1), (rows0, rows1), (st0, st1)
        semis, semgs, semss = (si0, si1), (sg0, sg1), (ss0, ss1)
        w = _wid()
        iota = lax.iota(jnp.int32, 16)

        def coords(i):
            unit = w * per_w + i
            sb = unit // 64
            rem = unit % 64
            return sb, rem // 2, rem % 2

        def fire_idx(i, p):
            sb, bt, h = coords(i)
            pltpu.async_copy(
                idx_hbm.at[sb, bt, pl.ds(h * 4, 4)], idxs[p], semis[p])

        def wait_idx(p):
            pltpu.make_async_copy(
                idx_hbm.at[0, 0, pl.ds(0, 4)], idxs[p], semis[p]).wait()

        def fire_gathers(p):
            for s8 in range(4):
                pltpu.async_copy(
                    wrm_hbm.at[idxs[p].at[s8]],
                    rowss[p].at[pl.ds(s8 * 128, 128)],
                    semgs[p])

        def wait_gathers(p):
            for s8 in range(4):
                pltpu.make_async_copy(
                    wrm_hbm.at[idxs[p].at[s8]],
                    rowss[p].at[pl.ds(s8 * 128, 128)],
                    semgs[p]).wait()

        skc = [[(iota + 16 * k) * 128 + s8 * 4096 for k in range(2)]
               for s8 in range(4)]

        def transpose(p):
            for s8 in range(4):
                def tbody(rb, c):
                    for half in range(2):
                        pairs = []
                        for dr in range(8):
                            b = rb * 16 + half * 8 + dr
                            b_vec = jnp.full((16,), b, jnp.int32)
                            r = s8 * 128 + b
                            for k in range(2):
                                vec = rowss[p][r, pl.ds(16 * k, 16)]
                                pairs.append((skc[s8][k] + b_vec, vec))
                        for idxv, vec in pairs:
                            plsc.store_scatter(stages[p], [idxv], vec)
                    return c

                lax.fori_loop(0, 8, tbody, 0)

        def fire_writes(i, p):
            sb, bt, h = coords(i)
            for s8 in range(4):
                for E in range(4):
                    pltpu.async_copy(
                        stages[p].at[pl.ds((s8 * 4 + E) * 1024, 1024)],
                        out_hbm.at[sb * 8 + h * 4 + s8, E, bt],
                        semss[p])

        def wait_writes(p):
            for s8 in range(4):
                for E in range(4):
                    pltpu.make_async_copy(
                        stages[p].at[pl.ds((s8 * 4 + E) * 1024, 1024)],
                        out_hbm.at[0, 0, 0], semss[p]).wait()

        def step(i, p):
            wait_gathers(p)

            @pl.when(i + 2 < per_w)
            def _():
                fire_idx(i + 2, p)

            @pl.when(i + 1 < per_w)
            def _():
                wait_idx(p ^ 1)
                fire_gathers(p ^ 1)

            @pl.when(i >= 2)
            def _():
                wait_writes(p)

            transpose(p)
            fire_writes(i, p)

        fire_idx(0, 0)
        fire_idx(1, 1)
        wait_idx(0)
        fire_gathers(0)

        def body(r, carry):
            step(2 * r, 0)
            step(2 * r + 1, 1)
            return carry

        lax.fori_loop(0, per_w // 2, body, 0)
        wait_writes(0)
        wait_writes(1)

    return kb(idx4, w_lin)


def kernel(input, weight):
    idx4 = (input.astype(jnp.int32).T
            .reshape(25, 8, 32, 128).transpose(0, 2, 1, 3))
    wtail = jnp.pad(weight[D_VOCAB - 64:, :].T, ((0, 0), (0, 64)))
    w_rm = _reformat_table(weight.T, wtail)
    w_lin = w_rm.reshape(WRM_ROWS * 4, D_EMB)
    out4 = _gather_native(idx4, w_lin)
    out5 = out4.reshape(200, 4, 32, 8, 128)
    return out5.transpose(2, 4, 0, 1, 3).reshape(4096, 200, 32)
